# baseline (device time: 32511 ns/iter reference)
import jax
import jax.numpy as jnp
from jax import lax
from jax.experimental import pallas as pl
from jax.experimental.pallas import tpu as pltpu

N_DEV = 4


def kernel(A, B):
    m_per, k = A.shape
    k2, n = B.shape
    assert k == k2
    m_half = m_per // 2
    TOP = pl.ds(0, m_half)
    BOT = pl.ds(m_half, m_half)

    def body(
        a_ref, b_ref, out_ref,
        my_bf, b_bf, recv_l, recv_r, recv_d, out_vmem,
        send_sems, recv_sems, copy_sems,
    ):
        my_pos = lax.axis_index("i")
        left = (my_pos - 1) % N_DEV
        right = (my_pos + 1) % N_DEV

        barrier_sem = pltpu.get_barrier_semaphore()
        for nbr in [left, right]:
            pl.semaphore_signal(
                barrier_sem, inc=1,
                device_id=(nbr,), device_id_type=pl.DeviceIdType.MESH,
            )
        my_bf[:, :] = a_ref[:, :].astype(jnp.bfloat16)
        pl.semaphore_wait(barrier_sem, 2)

        def rdma(i, src, dst, dev):
            return pltpu.make_async_remote_copy(
                src_ref=src, dst_ref=dst,
                send_sem=send_sems.at[i], recv_sem=recv_sems.at[i],
                device_id=(dev,), device_id_type=pl.DeviceIdType.MESH,
            )

        s_rt = rdma(0, my_bf.at[TOP, :], recv_l.at[TOP, :], right)
        s_lb = rdma(1, my_bf.at[BOT, :], recv_r.at[BOT, :], left)
        s_rb = rdma(2, my_bf.at[BOT, :], recv_l.at[BOT, :], right)
        s_lt = rdma(3, my_bf.at[TOP, :], recv_r.at[TOP, :], left)
        s_rt.start()
        s_lb.start()
        s_rb.start()
        s_lt.start()

        b_bf[:, :] = b_ref[:, :].astype(jnp.bfloat16)

        def mm(slot, rows, chunk_rows):
            out_vmem[slot, rows, :] = jnp.dot(
                chunk_rows, b_bf[:, :], preferred_element_type=jnp.float32
            )

        def store_half(sem_i, slot, rows, origin, row_off):
            copy = pltpu.make_async_copy(
                out_vmem.at[slot, rows, :],
                out_ref.at[pl.ds(origin * m_per + row_off, m_half), :],
                copy_sems.at[sem_i],
            )
            copy.start()
            return copy

        mm(0, TOP, my_bf[TOP, :])
        c0 = store_half(0, 0, TOP, my_pos, 0)
        mm(0, BOT, my_bf[BOT, :])
        c1 = store_half(1, 0, BOT, my_pos, m_half)

        m_q = m_per // 4
        QTR = [pl.ds(i * m_q, m_q) for i in range(4)]
        s_rt.wait_recv()
        f_r0 = rdma(4, recv_l.at[QTR[0], :], recv_d.at[QTR[0], :], right)
        f_r1 = rdma(5, recv_l.at[QTR[1], :], recv_d.at[QTR[1], :], right)
        f_r0.start()
        f_r1.start()
        mm(1, TOP, recv_l[TOP, :])
        c2 = store_half(2, 1, TOP, left, 0)

        s_lb.wait_recv()
        f_l0 = rdma(6, recv_r.at[QTR[2], :], recv_d.at[QTR[2], :], left)
        f_l1 = rdma(7, recv_r.at[QTR[3], :], recv_d.at[QTR[3], :], left)
        f_l0.start()
        f_l1.start()
        mm(2, BOT, recv_r[BOT, :])
        c3 = store_half(3, 2, BOT, right, m_half)

        s_rb.wait_recv()
        mm(1, BOT, recv_l[BOT, :])
        c4 = store_half(4, 1, BOT, left, m_half)
        s_lt.wait_recv()
        mm(2, TOP, recv_r[TOP, :])
        c5 = store_half(5, 2, TOP, right, 0)

        diag = (my_pos + 2) % N_DEV

        def store_qtr(sem_i, rows, row_off):
            copy = pltpu.make_async_copy(
                out_vmem.at[3, rows, :],
                out_ref.at[pl.ds(diag * m_per + row_off, m_q), :],
                copy_sems.at[sem_i],
            )
            copy.start()
            return copy

        cq = []
        for qi, f in enumerate([f_r0, f_r1, f_l0, f_l1]):
            f.wait_recv()
            mm(3, QTR[qi], recv_d[QTR[qi], :])
            cq.append(store_qtr(6 + qi, QTR[qi], qi * m_q))

        for c in [c0, c1, c2, c3, c4, c5] + cq:
            c.wait()
        for s in [s_rt, s_lb, s_rb, s_lt, f_r0, f_r1, f_l0, f_l1]:
            s.wait_send()

    return pl.pallas_call(
        body,
        out_shape=jax.ShapeDtypeStruct((N_DEV * m_per, n), jnp.float32),
        in_specs=[
            pl.BlockSpec(memory_space=pltpu.VMEM),
            pl.BlockSpec(memory_space=pltpu.VMEM),
        ],
        out_specs=pl.BlockSpec(memory_space=pl.ANY),
        scratch_shapes=[
            pltpu.VMEM((m_per, k), jnp.bfloat16),
            pltpu.VMEM((k, n), jnp.bfloat16),
            pltpu.VMEM((m_per, k), jnp.bfloat16),
            pltpu.VMEM((m_per, k), jnp.bfloat16),
            pltpu.VMEM((m_per, k), jnp.bfloat16),
            pltpu.VMEM((N_DEV, m_per, n), jnp.float32),
            pltpu.SemaphoreType.DMA((8,)),
            pltpu.SemaphoreType.DMA((8,)),
            pltpu.SemaphoreType.DMA((10,)),
        ],
        compiler_params=pltpu.CompilerParams(collective_id=0),
    )(A, B)


# device time: 32461 ns/iter; 1.0015x vs baseline; 1.0015x over previous
import jax
import jax.numpy as jnp
from jax import lax
from jax.experimental import pallas as pl
from jax.experimental.pallas import tpu as pltpu

N_DEV = 4


def kernel(A, B):
    m_per, k = A.shape
    k2, n = B.shape
    assert k == k2
    m_half = m_per // 2
    TOP = pl.ds(0, m_half)
    BOT = pl.ds(m_half, m_half)

    def body(
        a_ref, b_ref, out_ref,
        my_bf, b_bf, recv_l, recv_r, recv_d, out_vmem,
        send_sems, recv_sems, copy_sems,
    ):
        my_pos = lax.axis_index("i")
        left = (my_pos - 1) % N_DEV
        right = (my_pos + 1) % N_DEV

        barrier_sem = pltpu.get_barrier_semaphore()
        for nbr in [left, right]:
            pl.semaphore_signal(
                barrier_sem, inc=1,
                device_id=(nbr,), device_id_type=pl.DeviceIdType.MESH,
            )
        my_bf[:, :] = a_ref[:, :].astype(jnp.bfloat16)
        pl.semaphore_wait(barrier_sem, 2)

        def rdma(i, src, dst, dev):
            return pltpu.make_async_remote_copy(
                src_ref=src, dst_ref=dst,
                send_sem=send_sems.at[i], recv_sem=recv_sems.at[i],
                device_id=(dev,), device_id_type=pl.DeviceIdType.MESH,
            )

        s_rt = rdma(0, my_bf.at[TOP, :], recv_l.at[TOP, :], right)
        s_lb = rdma(1, my_bf.at[BOT, :], recv_r.at[BOT, :], left)
        s_rb = rdma(2, my_bf.at[BOT, :], recv_l.at[BOT, :], right)
        s_lt = rdma(3, my_bf.at[TOP, :], recv_r.at[TOP, :], left)
        s_rt.start()
        s_lb.start()
        s_rb.start()
        s_lt.start()

        b_bf[:, :] = b_ref[:, :].astype(jnp.bfloat16)

        def mm(slot, rows, chunk_rows):
            out_vmem[slot, rows, :] = jnp.dot(
                chunk_rows, b_bf[:, :], preferred_element_type=jnp.float32
            )

        def store_half(sem_i, slot, rows, origin, row_off):
            copy = pltpu.make_async_copy(
                out_vmem.at[slot, rows, :],
                out_ref.at[pl.ds(origin * m_per + row_off, m_half), :],
                copy_sems.at[sem_i],
            )
            copy.start()
            return copy

        mm(0, TOP, my_bf[TOP, :])
        c0 = store_half(0, 0, TOP, my_pos, 0)
        mm(0, BOT, my_bf[BOT, :])
        c1 = store_half(1, 0, BOT, my_pos, m_half)

        s_rt.wait_recv()
        f_r = rdma(4, recv_l.at[TOP, :], recv_d.at[TOP, :], right)
        f_r.start()
        mm(1, TOP, recv_l[TOP, :])
        c2 = store_half(2, 1, TOP, left, 0)

        s_lb.wait_recv()
        f_l = rdma(5, recv_r.at[BOT, :], recv_d.at[BOT, :], left)
        f_l.start()
        mm(2, BOT, recv_r[BOT, :])
        c3 = store_half(3, 2, BOT, right, m_half)

        s_rb.wait_recv()
        mm(1, BOT, recv_l[BOT, :])
        c4 = store_half(4, 1, BOT, left, m_half)
        s_lt.wait_recv()
        mm(2, TOP, recv_r[TOP, :])
        c5 = store_half(5, 2, TOP, right, 0)

        diag = (my_pos + 2) % N_DEV
        f_r.wait_recv()
        mm(3, TOP, recv_d[TOP, :])
        c6 = store_half(6, 3, TOP, diag, 0)
        f_l.wait_recv()
        mm(3, BOT, recv_d[BOT, :])
        c7 = store_half(7, 3, BOT, diag, m_half)

        for c in [c0, c1, c2, c3, c4, c5, c6, c7]:
            c.wait()
        for s in [s_rt, s_lb, s_rb, s_lt, f_r, f_l]:
            s.wait_send()

    return pl.pallas_call(
        body,
        out_shape=jax.ShapeDtypeStruct((N_DEV * m_per, n), jnp.float32),
        in_specs=[
            pl.BlockSpec(memory_space=pltpu.VMEM),
            pl.BlockSpec(memory_space=pltpu.VMEM),
        ],
        out_specs=pl.BlockSpec(memory_space=pl.ANY),
        scratch_shapes=[
            pltpu.VMEM((m_per, k), jnp.bfloat16),
            pltpu.VMEM((k, n), jnp.bfloat16),
            pltpu.VMEM((m_per, k), jnp.bfloat16),
            pltpu.VMEM((m_per, k), jnp.bfloat16),
            pltpu.VMEM((m_per, k), jnp.bfloat16),
            pltpu.VMEM((N_DEV, m_per, n), jnp.float32),
            pltpu.SemaphoreType.DMA((6,)),
            pltpu.SemaphoreType.DMA((6,)),
            pltpu.SemaphoreType.DMA((8,)),
        ],
        compiler_params=pltpu.CompilerParams(collective_id=0),
    )(A, B)
